# baseline (speedup 1.0000x reference)
import jax
import jax.numpy as jnp
import numpy as np
from jax import lax
from jax.experimental import pallas as pl
from jax.experimental.pallas import tpu as pltpu

B = 8
C = 19
H = 512
W = 512
P = H * W
N = B * P
KK = 100000
THRESH = 0.7
THRESH_BITS = int(np.float32(THRESH).view(np.int32))
ONE_BITS = int(np.float32(1.0).view(np.int32))

RB = 512
NG = H // RB


def _tree(op, vals):
    while len(vals) > 1:
        nxt = [op(vals[i], vals[i + 1]) for i in range(0, len(vals) - 1, 2)]
        if len(vals) % 2:
            nxt.append(vals[-1])
        vals = nxt
    return vals[0]


def _ce_et_s(x, t):
    xs = [x[c] for c in range(C)]
    m = _tree(jnp.maximum, xs)
    s = _tree(jnp.add, [jnp.exp(xc - m) for xc in xs])
    st = _tree(jnp.add,
               [jnp.where(t == c, xs[c], jnp.float32(0.0)) for c in range(C)])
    ce = (m + jnp.log(s)) - st
    et = jnp.exp(st - m)
    return ce, et, s


def _fused_kernel(score_ref, target_ref, s7_ref, c7_ref, acc_s, acc_c):
    b = pl.program_id(0)
    g = pl.program_id(1)

    @pl.when((b == 0) & (g == 0))
    def _init():
        acc_s[...] = jnp.zeros((RB, W), jnp.float32)
        acc_c[...] = jnp.zeros((RB, W), jnp.float32)

    ce, et, s = _ce_et_s(score_ref[0], target_ref[0])
    keep = et < jnp.float32(THRESH) * s
    acc_s[...] += jnp.where(keep, ce, jnp.float32(0.0))
    acc_c[...] += keep.astype(jnp.float32)

    @pl.when((b == B - 1) & (g == NG - 1))
    def _finish():
        s7_ref[...] = jnp.sum(acc_s[...])[None, None]
        c7_ref[...] = jnp.sum(acc_c[...])[None, None]


def _ce_pg_kernel(score_ref, target_ref, ce_ref, pg_ref):
    ce, et, s = _ce_et_s(score_ref[0], target_ref[0])
    ce_ref[0] = ce
    pg_ref[0] = et / s


SEL_ROWS = 64
SEL_CHUNK = 8
SEL_ITERS = 19
SEL_W = N // SEL_ROWS


def _select_kernel(pg_ref, ce_ref, out_ref):
    nchunks = SEL_ROWS // SEL_CHUNK

    def count_le(v):
        def body(j, acc):
            blk = lax.bitcast_convert_type(
                pg_ref[pl.ds(j * SEL_CHUNK, SEL_CHUNK), :], jnp.int32)
            return acc + (blk <= v).astype(jnp.int32)
        acc = lax.fori_loop(
            0, nchunks, body, jnp.zeros((SEL_CHUNK, SEL_W), jnp.int32))
        return jnp.sum(acc)

    c7 = count_le(jnp.int32(THRESH_BITS - 1))

    def bisect(_, carry):
        lo, hi = carry
        mid = lo + (hi - lo) // 2
        big = count_le(mid) >= (KK + 1)
        new_lo = jnp.where(big, lo, mid)
        new_hi = jnp.where(big, mid, hi)
        done = (hi - lo) <= 1
        return (jnp.where(done, lo, new_lo), jnp.where(done, hi, new_hi))

    lo0 = jnp.int32(THRESH_BITS - 1)
    hi0 = jnp.int32(ONE_BITS)
    _, kth_bits = lax.fori_loop(0, SEL_ITERS, bisect, (lo0, hi0))

    thr_bits = jnp.where(c7 >= (KK + 1), jnp.int32(THRESH_BITS), kth_bits)

    def final_body(j, carry):
        s_acc, c_acc = carry
        sl = pl.ds(j * SEL_CHUNK, SEL_CHUNK)
        blk = lax.bitcast_convert_type(pg_ref[sl, :], jnp.int32)
        keep = (blk < thr_bits).astype(jnp.float32)
        return (s_acc + ce_ref[sl, :] * keep, c_acc + keep)

    z = jnp.zeros((SEL_CHUNK, SEL_W), jnp.float32)
    s_acc, c_acc = lax.fori_loop(0, nchunks, final_body, (z, z))
    loss = jnp.sum(s_acc) / jnp.maximum(jnp.sum(c_acc), jnp.float32(1.0))
    out_ref[...] = loss[None, None]


@jax.jit
def kernel(score, target):
    grid = (B, NG)
    in_specs = [
        pl.BlockSpec((1, C, RB, W), lambda b, g: (b, 0, g, 0)),
        pl.BlockSpec((1, RB, W), lambda b, g: (b, g, 0)),
    ]

    s7, c7 = pl.pallas_call(
        _fused_kernel,
        grid=grid,
        in_specs=in_specs,
        out_specs=[
            pl.BlockSpec((1, 1), lambda b, g: (0, 0)),
            pl.BlockSpec((1, 1), lambda b, g: (0, 0)),
        ],
        out_shape=[
            jax.ShapeDtypeStruct((1, 1), jnp.float32),
            jax.ShapeDtypeStruct((1, 1), jnp.float32),
        ],
        scratch_shapes=[
            pltpu.VMEM((RB, W), jnp.float32),
            pltpu.VMEM((RB, W), jnp.float32),
        ],
    )(score, target)
    s7 = s7[0, 0]
    c7 = c7[0, 0]

    def common_case():
        return s7 / jnp.maximum(c7, jnp.float32(1.0))

    def rare_case():
        ce, pg = pl.pallas_call(
            _ce_pg_kernel,
            grid=grid,
            in_specs=in_specs,
            out_specs=[
                pl.BlockSpec((1, RB, W), lambda b, g: (b, g, 0)),
                pl.BlockSpec((1, RB, W), lambda b, g: (b, g, 0)),
            ],
            out_shape=[
                jax.ShapeDtypeStruct((B, H, W), jnp.float32),
                jax.ShapeDtypeStruct((B, H, W), jnp.float32),
            ],
        )(score, target)
        out = pl.pallas_call(
            _select_kernel,
            out_shape=jax.ShapeDtypeStruct((1, 1), jnp.float32),
        )(pg.reshape(SEL_ROWS, SEL_W), ce.reshape(SEL_ROWS, SEL_W))
        return out[0, 0]

    return lax.cond(c7 >= jnp.float32(KK + 1), common_case, rare_case)
